# baseline (speedup 1.0000x reference)
import functools

import jax
import jax.numpy as jnp
from jax import lax
from jax.experimental import pallas as pl
from jax.experimental.pallas import tpu as pltpu
from jax.experimental.pallas import tpu_sc as plsc

N_NODES = 10000
N_EDGES = 320000
D = 128
NUM_GRAPHS = 64

NC, NS, L = 2, 16, 16
NW = NC * NS
CHUNK = 128
N_CHUNKS = 2560
C0 = 160
C1 = 0
IB = 32
E_PAD = N_CHUNKS * CHUNK
ROWS_PER_TILE = 632
ACC_ROWS = ROWS_PER_TILE * NS
DUMP_ROW = N_NODES

BLK = 1000
NBLK = N_NODES // BLK


def _scale_body(x_ref, norm_ref, o_ref):
    o_ref[...] = x_ref[...] * norm_ref[...]


def _edge_agg_body(h1_hbm, src_hbm, dst_hbm, out_hbm,
                   src_v, dst_v, rows_v, zero_v, acc_sh,
                   gsem0, gsem1, ssem0, ssem1):
    c = lax.axis_index("c")
    s = lax.axis_index("s")

    def zero_acc():
        zvec = jnp.zeros((L,), jnp.float32)
        for r in range(16):
            for q in range(D // L):
                zero_v[r, pl.ds(q * L, L)] = zvec

        def zbody(i, carry):
            pltpu.sync_copy(zero_v,
                            acc_sh.at[pl.ds(s * ROWS_PER_TILE + i * 16, 16)])
            return carry
        lax.fori_loop(0, ROWS_PER_TILE // 16, zbody, 0)
        pltpu.sync_copy(
            zero_v.at[pl.ds(0, ROWS_PER_TILE % 16)],
            acc_sh.at[pl.ds(s * ROWS_PER_TILE + 16 * (ROWS_PER_TILE // 16),
                            ROWS_PER_TILE % 16)])

    def fire_gather(g, p, sem):
        pltpu.async_copy(h1_hbm.at[src_v.at[g]], rows_v.at[p], sem)

    def drain_gather(p, sem):
        pltpu.make_async_copy(h1_hbm.at[pl.ds(0, CHUNK)],
                              rows_v.at[p], sem).wait()

    def fire_scatter(g, p, sem):
        pltpu.async_copy(rows_v.at[p], acc_sh.at[dst_v.at[g]], sem, add=True)

    def drain_scatter(p, sem):
        pltpu.make_async_copy(rows_v.at[p],
                              acc_sh.at[pl.ds(0, CHUNK)], sem).wait()

    def run_round(base):
        pltpu.sync_copy(src_hbm.at[pl.ds(base, IB)], src_v)
        pltpu.sync_copy(dst_hbm.at[pl.ds(base, IB)], dst_v)

        fire_gather(0, 0, gsem0)
        fire_gather(1, 1, gsem1)

        def body(i, carry):
            g0 = 2 * i
            drain_gather(0, gsem0)
            fire_scatter(g0, 0, ssem0)
            drain_gather(1, gsem1)
            drain_scatter(0, ssem0)

            @pl.when(i < IB // 2 - 1)
            def _():
                fire_gather(g0 + 2, 0, gsem0)

            fire_scatter(g0 + 1, 1, ssem1)
            drain_scatter(1, ssem1)

            @pl.when(i < IB // 2 - 1)
            def _():
                fire_gather(g0 + 3, 1, gsem1)

            return carry
        lax.fori_loop(0, IB // 2, body, 0)

    @pl.when(c == 0)
    def _():
        zero_acc()
        plsc.subcore_barrier()
        for r in range(C0 // IB):
            run_round(s * C0 + r * IB)
        plsc.subcore_barrier()
        pltpu.sync_copy(acc_sh.at[pl.ds(s * ROWS_PER_TILE, ROWS_PER_TILE)],
                        out_hbm.at[0, pl.ds(s * ROWS_PER_TILE, ROWS_PER_TILE)])


def _fc_body(p_ref, norm_ref, w_ref, b_ref, h3_ref, sum_ref, sumsq_ref):
    i = pl.program_id(0)
    h = p_ref[0] * norm_ref[...]
    h = lax.dot_general(h, w_ref[...], (((1,), (1,)), ((), ())),
                        precision=lax.Precision.HIGHEST,
                        preferred_element_type=jnp.float32)
    h = jnp.maximum(h + b_ref[...], 0.0)
    h3_ref[...] = h

    @pl.when(i == 0)
    def _():
        sum_ref[...] = jnp.zeros_like(sum_ref)
        sumsq_ref[...] = jnp.zeros_like(sumsq_ref)

    sum_ref[...] += jnp.sum(h, axis=0, keepdims=True)
    sumsq_ref[...] += jnp.sum(h * h, axis=0, keepdims=True)


def _bn_body(h3_ref, sum_ref, sumsq_ref, gamma_ref, beta_ref, gid_ref,
             hbn_ref, phis_ref):
    i = pl.program_id(0)
    inv_n = 1.0 / N_NODES
    mean = sum_ref[...] * inv_n
    var = sumsq_ref[...] * inv_n - mean * mean
    scale = gamma_ref[...] / jnp.sqrt(var + 1e-5)
    hbn = (h3_ref[...] - mean) * scale + beta_ref[...]
    hbn_ref[...] = hbn

    gid = gid_ref[0]
    gids = lax.broadcasted_iota(jnp.int32, (NUM_GRAPHS, BLK), 0)
    onehot = (gids == gid).astype(jnp.float32)
    contrib = lax.dot_general(onehot, hbn, (((1,), (0,)), ((), ())),
                              precision=lax.Precision.HIGHEST,
                              preferred_element_type=jnp.float32)

    @pl.when(i == 0)
    def _():
        phis_ref[...] = jnp.zeros_like(phis_ref)

    phis_ref[...] += contrib


def kernel(x, edge_index, norm, graph_ids, W, b, gamma, beta):
    h1 = pl.pallas_call(
        _scale_body,
        grid=(NBLK,),
        in_specs=[pl.BlockSpec((BLK, D), lambda i: (i, 0)),
                  pl.BlockSpec((BLK, 1), lambda i: (i, 0))],
        out_specs=pl.BlockSpec((BLK, D), lambda i: (i, 0)),
        out_shape=jax.ShapeDtypeStruct((N_NODES, D), jnp.float32),
    )(x, norm)

    pad = E_PAD - N_EDGES
    src2d = jnp.concatenate(
        [edge_index[0], jnp.zeros((pad,), jnp.int32)]).reshape(-1, CHUNK)
    dump = DUMP_ROW + jnp.arange(pad, dtype=jnp.int32) % (ACC_ROWS - N_NODES)
    dst2d = jnp.concatenate([edge_index[1], dump]).reshape(-1, CHUNK)

    mesh = plsc.VectorSubcoreMesh(core_axis_name="c", subcore_axis_name="s",
                                  num_cores=NC, num_subcores=NS)
    partials = pl.kernel(
        _edge_agg_body,
        out_type=jax.ShapeDtypeStruct((1, ACC_ROWS, D), jnp.float32),
        mesh=mesh,
        scratch_types=[
            pltpu.VMEM((IB, CHUNK), jnp.int32),
            pltpu.VMEM((IB, CHUNK), jnp.int32),
            pltpu.VMEM((2, CHUNK, D), jnp.float32),
            pltpu.VMEM((16, D), jnp.float32),
            pltpu.VMEM_SHARED((ACC_ROWS, D), jnp.float32),
            pltpu.SemaphoreType.DMA,
            pltpu.SemaphoreType.DMA,
            pltpu.SemaphoreType.DMA,
            pltpu.SemaphoreType.DMA,
        ],
    )(h1, src2d, dst2d)

    p = partials[:, :N_NODES, :]

    h3, colsum, colsumsq = pl.pallas_call(
        _fc_body,
        grid=(NBLK,),
        in_specs=[pl.BlockSpec((1, BLK, D), lambda i: (0, i, 0)),
                  pl.BlockSpec((BLK, 1), lambda i: (i, 0)),
                  pl.BlockSpec((D, D), lambda i: (0, 0)),
                  pl.BlockSpec((1, D), lambda i: (0, 0))],
        out_specs=[pl.BlockSpec((BLK, D), lambda i: (i, 0)),
                   pl.BlockSpec((1, D), lambda i: (0, 0)),
                   pl.BlockSpec((1, D), lambda i: (0, 0))],
        out_shape=[jax.ShapeDtypeStruct((N_NODES, D), jnp.float32),
                   jax.ShapeDtypeStruct((1, D), jnp.float32),
                   jax.ShapeDtypeStruct((1, D), jnp.float32)],
    )(p, norm, W, b.reshape(1, D))

    gid3 = graph_ids.reshape(NBLK, 1, BLK)
    hbn, phis = pl.pallas_call(
        _bn_body,
        grid=(NBLK,),
        in_specs=[pl.BlockSpec((BLK, D), lambda i: (i, 0)),
                  pl.BlockSpec((1, D), lambda i: (0, 0)),
                  pl.BlockSpec((1, D), lambda i: (0, 0)),
                  pl.BlockSpec((1, D), lambda i: (0, 0)),
                  pl.BlockSpec((1, D), lambda i: (0, 0)),
                  pl.BlockSpec((1, 1, BLK), lambda i: (i, 0, 0))],
        out_specs=[pl.BlockSpec((BLK, D), lambda i: (i, 0)),
                   pl.BlockSpec((NUM_GRAPHS, D), lambda i: (0, 0))],
        out_shape=[jax.ShapeDtypeStruct((N_NODES, D), jnp.float32),
                   jax.ShapeDtypeStruct((NUM_GRAPHS, D), jnp.float32)],
    )(h3, colsum, colsumsq, gamma.reshape(1, D), beta.reshape(1, D), gid3)

    return (hbn, phis)


# speedup vs baseline: 3.5890x; 3.5890x over previous
import functools

import jax
import jax.numpy as jnp
from jax import lax
from jax.experimental import pallas as pl
from jax.experimental.pallas import tpu as pltpu
from jax.experimental.pallas import tpu_sc as plsc

N_NODES = 10000
N_EDGES = 320000
D = 128
NUM_GRAPHS = 64

NC, NS, L = 2, 16, 16
NW = NC * NS
CHUNK = 125
N_CHUNKS = N_EDGES // CHUNK
CPW = N_CHUNKS // NW
IB = 40
ROWS_PER_TILE = 632
ACC_ROWS = ROWS_PER_TILE * NS

BLK = 1000
NBLK = N_NODES // BLK


def _scale_body(x_ref, norm_ref, o_ref):
    o_ref[...] = x_ref[...] * norm_ref[...]


def _edge_agg_body(h1_hbm, src_hbm, dst_hbm, out_hbm,
                   src_v, dst_v, rows_v, zero_v, acc_sh,
                   gsem0, gsem1, ssem0, ssem1):
    c = lax.axis_index("c")
    s = lax.axis_index("s")

    def zero_acc():
        zvec = jnp.zeros((L,), jnp.float32)
        for r in range(16):
            for q in range(D // L):
                zero_v[r, pl.ds(q * L, L)] = zvec

        def zbody(i, carry):
            pltpu.sync_copy(zero_v,
                            acc_sh.at[pl.ds(s * ROWS_PER_TILE + i * 16, 16)])
            return carry
        lax.fori_loop(0, ROWS_PER_TILE // 16, zbody, 0)
        pltpu.sync_copy(
            zero_v.at[pl.ds(0, ROWS_PER_TILE % 16)],
            acc_sh.at[pl.ds(s * ROWS_PER_TILE + 16 * (ROWS_PER_TILE // 16),
                            ROWS_PER_TILE % 16)])

    def fire_gather(g, p, sem):
        pltpu.async_copy(h1_hbm.at[src_v.at[g]], rows_v.at[p], sem)

    def drain_gather(g, p, sem):
        pltpu.make_async_copy(h1_hbm.at[src_v.at[g]],
                              rows_v.at[p], sem).wait()

    def fire_scatter(g, p, sem):
        pltpu.async_copy(rows_v.at[p], acc_sh.at[dst_v.at[g]], sem, add=True)

    def drain_scatter(g, p, sem):
        pltpu.make_async_copy(rows_v.at[p],
                              acc_sh.at[dst_v.at[g]], sem).wait()

    def run_round(base):
        pltpu.sync_copy(src_hbm.at[pl.ds(base, IB)], src_v)
        pltpu.sync_copy(dst_hbm.at[pl.ds(base, IB)], dst_v)

        fire_gather(0, 0, gsem0)
        fire_gather(1, 1, gsem1)

        def body(i, carry):
            g0 = 2 * i
            drain_gather(g0, 0, gsem0)
            fire_scatter(g0, 0, ssem0)
            drain_gather(g0 + 1, 1, gsem1)
            drain_scatter(g0, 0, ssem0)

            @pl.when(i < IB // 2 - 1)
            def _():
                fire_gather(g0 + 2, 0, gsem0)

            fire_scatter(g0 + 1, 1, ssem1)
            drain_scatter(g0 + 1, 1, ssem1)

            @pl.when(i < IB // 2 - 1)
            def _():
                fire_gather(g0 + 3, 1, gsem1)

            return carry
        lax.fori_loop(0, IB // 2, body, 0)

    zero_acc()
    plsc.subcore_barrier()
    wid = c * NS + s
    for r in range(CPW // IB):
        run_round(wid * CPW + r * IB)
    plsc.subcore_barrier()
    pltpu.sync_copy(acc_sh.at[pl.ds(s * ROWS_PER_TILE, ROWS_PER_TILE)],
                    out_hbm.at[c, pl.ds(s * ROWS_PER_TILE, ROWS_PER_TILE)])


def _fc_body(p_ref, norm_ref, w_ref, b_ref, h3_ref, sum_ref, sumsq_ref):
    i = pl.program_id(0)
    h = (p_ref[0] + p_ref[1]) * norm_ref[...]
    h = lax.dot_general(h, w_ref[...], (((1,), (1,)), ((), ())),
                        precision=lax.Precision.HIGHEST,
                        preferred_element_type=jnp.float32)
    h = jnp.maximum(h + b_ref[...], 0.0)
    h3_ref[...] = h

    @pl.when(i == 0)
    def _():
        sum_ref[...] = jnp.zeros_like(sum_ref)
        sumsq_ref[...] = jnp.zeros_like(sumsq_ref)

    sum_ref[...] += jnp.sum(h, axis=0, keepdims=True)
    sumsq_ref[...] += jnp.sum(h * h, axis=0, keepdims=True)


def _bn_body(h3_ref, sum_ref, sumsq_ref, gamma_ref, beta_ref, gid_ref,
             hbn_ref, phis_ref):
    i = pl.program_id(0)
    inv_n = 1.0 / N_NODES
    mean = sum_ref[...] * inv_n
    var = sumsq_ref[...] * inv_n - mean * mean
    scale = gamma_ref[...] / jnp.sqrt(var + 1e-5)
    hbn = (h3_ref[...] - mean) * scale + beta_ref[...]
    hbn_ref[...] = hbn

    gid = gid_ref[0]
    gids = lax.broadcasted_iota(jnp.int32, (NUM_GRAPHS, BLK), 0)
    onehot = (gids == gid).astype(jnp.float32)
    contrib = lax.dot_general(onehot, hbn, (((1,), (0,)), ((), ())),
                              precision=lax.Precision.HIGHEST,
                              preferred_element_type=jnp.float32)

    @pl.when(i == 0)
    def _():
        phis_ref[...] = jnp.zeros_like(phis_ref)

    phis_ref[...] += contrib


def kernel(x, edge_index, norm, graph_ids, W, b, gamma, beta):
    h1 = pl.pallas_call(
        _scale_body,
        grid=(NBLK,),
        in_specs=[pl.BlockSpec((BLK, D), lambda i: (i, 0)),
                  pl.BlockSpec((BLK, 1), lambda i: (i, 0))],
        out_specs=pl.BlockSpec((BLK, D), lambda i: (i, 0)),
        out_shape=jax.ShapeDtypeStruct((N_NODES, D), jnp.float32),
    )(x, norm)

    src2d = edge_index[0].reshape(N_CHUNKS, CHUNK)
    dst2d = edge_index[1].reshape(N_CHUNKS, CHUNK)

    mesh = plsc.VectorSubcoreMesh(core_axis_name="c", subcore_axis_name="s",
                                  num_cores=NC, num_subcores=NS)
    partials = pl.kernel(
        _edge_agg_body,
        out_type=jax.ShapeDtypeStruct((NC, ACC_ROWS, D), jnp.float32),
        mesh=mesh,
        scratch_types=[
            pltpu.VMEM((IB, CHUNK), jnp.int32),
            pltpu.VMEM((IB, CHUNK), jnp.int32),
            pltpu.VMEM((2, CHUNK, D), jnp.float32),
            pltpu.VMEM((16, D), jnp.float32),
            pltpu.VMEM_SHARED((ACC_ROWS, D), jnp.float32),
            pltpu.SemaphoreType.DMA,
            pltpu.SemaphoreType.DMA,
            pltpu.SemaphoreType.DMA,
            pltpu.SemaphoreType.DMA,
        ],
    )(h1, src2d, dst2d)

    p = partials[:, :N_NODES, :]

    h3, colsum, colsumsq = pl.pallas_call(
        _fc_body,
        grid=(NBLK,),
        in_specs=[pl.BlockSpec((NC, BLK, D), lambda i: (0, i, 0)),
                  pl.BlockSpec((BLK, 1), lambda i: (i, 0)),
                  pl.BlockSpec((D, D), lambda i: (0, 0)),
                  pl.BlockSpec((1, D), lambda i: (0, 0))],
        out_specs=[pl.BlockSpec((BLK, D), lambda i: (i, 0)),
                   pl.BlockSpec((1, D), lambda i: (0, 0)),
                   pl.BlockSpec((1, D), lambda i: (0, 0))],
        out_shape=[jax.ShapeDtypeStruct((N_NODES, D), jnp.float32),
                   jax.ShapeDtypeStruct((1, D), jnp.float32),
                   jax.ShapeDtypeStruct((1, D), jnp.float32)],
    )(p, norm, W, b.reshape(1, D))

    gid3 = graph_ids.reshape(NBLK, 1, BLK)
    hbn, phis = pl.pallas_call(
        _bn_body,
        grid=(NBLK,),
        in_specs=[pl.BlockSpec((BLK, D), lambda i: (i, 0)),
                  pl.BlockSpec((1, D), lambda i: (0, 0)),
                  pl.BlockSpec((1, D), lambda i: (0, 0)),
                  pl.BlockSpec((1, D), lambda i: (0, 0)),
                  pl.BlockSpec((1, D), lambda i: (0, 0)),
                  pl.BlockSpec((1, 1, BLK), lambda i: (i, 0, 0))],
        out_specs=[pl.BlockSpec((BLK, D), lambda i: (i, 0)),
                   pl.BlockSpec((NUM_GRAPHS, D), lambda i: (0, 0))],
        out_shape=[jax.ShapeDtypeStruct((N_NODES, D), jnp.float32),
                   jax.ShapeDtypeStruct((NUM_GRAPHS, D), jnp.float32)],
    )(h3, colsum, colsumsq, gamma.reshape(1, D), beta.reshape(1, D), gid3)

    return (hbn, phis)


# speedup vs baseline: 3.9659x; 1.1050x over previous
import functools

import jax
import jax.numpy as jnp
from jax import lax
from jax.experimental import pallas as pl
from jax.experimental.pallas import tpu as pltpu
from jax.experimental.pallas import tpu_sc as plsc

N_NODES = 10000
N_EDGES = 320000
D = 128
NUM_GRAPHS = 64

NC, NS, L = 2, 16, 16
NW = NC * NS
CHUNK = 125
N_CHUNKS = N_EDGES // CHUNK
CPW = N_CHUNKS // NW
IB = 40
ROWS_PER_TILE = 632
ACC_ROWS = ROWS_PER_TILE * NS

BLK = 1000
NBLK = N_NODES // BLK


def _scale_body(x_ref, norm_ref, o_ref):
    o_ref[...] = x_ref[...] * norm_ref[...]


def _edge_agg_body(h1_hbm, edges_hbm, out_hbm,
                   src_v, dst_v, rows_v, zero_v, acc_sh,
                   gsem0, gsem1, ssem0, ssem1):
    c = lax.axis_index("c")
    s = lax.axis_index("s")

    def zero_acc():
        zvec = jnp.zeros((L,), jnp.float32)
        for r in range(16):
            for q in range(D // L):
                zero_v[r, pl.ds(q * L, L)] = zvec

        def zbody(i, carry):
            pltpu.sync_copy(zero_v,
                            acc_sh.at[pl.ds(s * ROWS_PER_TILE + i * 16, 16)])
            return carry
        lax.fori_loop(0, ROWS_PER_TILE // 16, zbody, 0)
        pltpu.sync_copy(
            zero_v.at[pl.ds(0, ROWS_PER_TILE % 16)],
            acc_sh.at[pl.ds(s * ROWS_PER_TILE + 16 * (ROWS_PER_TILE // 16),
                            ROWS_PER_TILE % 16)])

    def fire_gather(g, p, sem):
        pltpu.async_copy(h1_hbm.at[src_v.at[g]], rows_v.at[p], sem)

    def drain_gather(g, p, sem):
        pltpu.make_async_copy(h1_hbm.at[src_v.at[g]],
                              rows_v.at[p], sem).wait()

    def fire_scatter(g, p, sem):
        pltpu.async_copy(rows_v.at[p], acc_sh.at[dst_v.at[g]], sem, add=True)

    def drain_scatter(g, p, sem):
        pltpu.make_async_copy(rows_v.at[p],
                              acc_sh.at[dst_v.at[g]], sem).wait()

    def run_round(base):
        pltpu.sync_copy(edges_hbm.at[0, pl.ds(base, IB)], src_v)
        pltpu.sync_copy(edges_hbm.at[1, pl.ds(base, IB)], dst_v)

        fire_gather(0, 0, gsem0)
        fire_gather(1, 1, gsem1)

        def body(i, carry):
            g0 = 2 * i
            drain_gather(g0, 0, gsem0)
            fire_scatter(g0, 0, ssem0)
            drain_gather(g0 + 1, 1, gsem1)
            drain_scatter(g0, 0, ssem0)

            @pl.when(i < IB // 2 - 1)
            def _():
                fire_gather(g0 + 2, 0, gsem0)

            fire_scatter(g0 + 1, 1, ssem1)
            drain_scatter(g0 + 1, 1, ssem1)

            @pl.when(i < IB // 2 - 1)
            def _():
                fire_gather(g0 + 3, 1, gsem1)

            return carry
        lax.fori_loop(0, IB // 2, body, 0)

    zero_acc()
    plsc.subcore_barrier()
    wid = c * NS + s
    for r in range(CPW // IB):
        run_round(wid * CPW + r * IB)
    plsc.subcore_barrier()
    pltpu.sync_copy(acc_sh.at[pl.ds(s * ROWS_PER_TILE, ROWS_PER_TILE)],
                    out_hbm.at[c, pl.ds(s * ROWS_PER_TILE, ROWS_PER_TILE)])


def _fc_body(p_ref, norm_ref, w_ref, b_ref, h3_ref, sum_ref, sumsq_ref):
    i = pl.program_id(0)
    h = (p_ref[0] + p_ref[1]) * norm_ref[...]
    h = lax.dot_general(h, w_ref[...], (((1,), (1,)), ((), ())),
                        precision=lax.Precision.HIGHEST,
                        preferred_element_type=jnp.float32)
    h = jnp.maximum(h + b_ref[...], 0.0)
    h3_ref[...] = h

    @pl.when(i == 0)
    def _():
        sum_ref[...] = jnp.zeros_like(sum_ref)
        sumsq_ref[...] = jnp.zeros_like(sumsq_ref)

    sum_ref[...] += jnp.sum(h, axis=0, keepdims=True)
    sumsq_ref[...] += jnp.sum(h * h, axis=0, keepdims=True)


def _bn_body(h3_ref, sum_ref, sumsq_ref, gamma_ref, beta_ref, gid_ref,
             hbn_ref, phis_ref):
    i = pl.program_id(0)
    inv_n = 1.0 / N_NODES
    mean = sum_ref[...] * inv_n
    var = sumsq_ref[...] * inv_n - mean * mean
    scale = gamma_ref[...] / jnp.sqrt(var + 1e-5)
    hbn = (h3_ref[...] - mean) * scale + beta_ref[...]
    hbn_ref[...] = hbn

    gid = gid_ref[0]
    gids = lax.broadcasted_iota(jnp.int32, (NUM_GRAPHS, BLK), 0)
    onehot = (gids == gid).astype(jnp.float32)
    contrib = lax.dot_general(onehot, hbn, (((1,), (0,)), ((), ())),
                              precision=lax.Precision.HIGHEST,
                              preferred_element_type=jnp.float32)

    @pl.when(i == 0)
    def _():
        phis_ref[...] = jnp.zeros_like(phis_ref)

    phis_ref[...] += contrib


def kernel(x, edge_index, norm, graph_ids, W, b, gamma, beta):
    h1 = pl.pallas_call(
        _scale_body,
        grid=(NBLK,),
        in_specs=[pl.BlockSpec((BLK, D), lambda i: (i, 0)),
                  pl.BlockSpec((BLK, 1), lambda i: (i, 0))],
        out_specs=pl.BlockSpec((BLK, D), lambda i: (i, 0)),
        out_shape=jax.ShapeDtypeStruct((N_NODES, D), jnp.float32),
    )(x, norm)

    edges3 = edge_index.reshape(2, N_CHUNKS, CHUNK)

    mesh = plsc.VectorSubcoreMesh(core_axis_name="c", subcore_axis_name="s",
                                  num_cores=NC, num_subcores=NS)
    partials = pl.kernel(
        _edge_agg_body,
        out_type=jax.ShapeDtypeStruct((NC, ACC_ROWS, D), jnp.float32),
        mesh=mesh,
        scratch_types=[
            pltpu.VMEM((IB, CHUNK), jnp.int32),
            pltpu.VMEM((IB, CHUNK), jnp.int32),
            pltpu.VMEM((2, CHUNK, D), jnp.float32),
            pltpu.VMEM((16, D), jnp.float32),
            pltpu.VMEM_SHARED((ACC_ROWS, D), jnp.float32),
            pltpu.SemaphoreType.DMA,
            pltpu.SemaphoreType.DMA,
            pltpu.SemaphoreType.DMA,
            pltpu.SemaphoreType.DMA,
        ],
    )(h1, edges3)

    h3, colsum, colsumsq = pl.pallas_call(
        _fc_body,
        grid=(NBLK,),
        in_specs=[pl.BlockSpec((NC, BLK, D), lambda i: (0, i, 0)),
                  pl.BlockSpec((BLK, 1), lambda i: (i, 0)),
                  pl.BlockSpec((D, D), lambda i: (0, 0)),
                  pl.BlockSpec((1, D), lambda i: (0, 0))],
        out_specs=[pl.BlockSpec((BLK, D), lambda i: (i, 0)),
                   pl.BlockSpec((1, D), lambda i: (0, 0)),
                   pl.BlockSpec((1, D), lambda i: (0, 0))],
        out_shape=[jax.ShapeDtypeStruct((N_NODES, D), jnp.float32),
                   jax.ShapeDtypeStruct((1, D), jnp.float32),
                   jax.ShapeDtypeStruct((1, D), jnp.float32)],
    )(partials, norm, W, b.reshape(1, D))

    gid3 = graph_ids.reshape(NBLK, 1, BLK)
    hbn, phis = pl.pallas_call(
        _bn_body,
        grid=(NBLK,),
        in_specs=[pl.BlockSpec((BLK, D), lambda i: (i, 0)),
                  pl.BlockSpec((1, D), lambda i: (0, 0)),
                  pl.BlockSpec((1, D), lambda i: (0, 0)),
                  pl.BlockSpec((1, D), lambda i: (0, 0)),
                  pl.BlockSpec((1, D), lambda i: (0, 0)),
                  pl.BlockSpec((1, 1, BLK), lambda i: (i, 0, 0))],
        out_specs=[pl.BlockSpec((BLK, D), lambda i: (i, 0)),
                   pl.BlockSpec((NUM_GRAPHS, D), lambda i: (0, 0))],
        out_shape=[jax.ShapeDtypeStruct((N_NODES, D), jnp.float32),
                   jax.ShapeDtypeStruct((NUM_GRAPHS, D), jnp.float32)],
    )(h3, colsum, colsumsq, gamma.reshape(1, D), beta.reshape(1, D), gid3)

    return (hbn, phis)


# speedup vs baseline: 4.0389x; 1.0184x over previous
import functools

import jax
import jax.numpy as jnp
from jax import lax
from jax.experimental import pallas as pl
from jax.experimental.pallas import tpu as pltpu
from jax.experimental.pallas import tpu_sc as plsc

N_NODES = 10000
N_EDGES = 320000
D = 128
NUM_GRAPHS = 64

NC, NS, L = 2, 16, 16
NW = NC * NS
CHUNK = 125
N_CHUNKS = N_EDGES // CHUNK
CPW = N_CHUNKS // NW
IB = 40
ROWS_PER_TILE = 632
ACC_ROWS = ROWS_PER_TILE * NS

BLK = 1000
NBLK = N_NODES // BLK


def _scale_body(x_ref, norm_ref, o_ref):
    o_ref[...] = x_ref[...] * norm_ref[...]


def _edge_agg_body(h1_hbm, edges_hbm, out_hbm,
                   src_v, dst_v, rows_v, zero_v, acc_sh,
                   gsem0, gsem1, ssem0, ssem1):
    c = lax.axis_index("c")
    s = lax.axis_index("s")

    def zero_acc():
        zvec = jnp.zeros((L,), jnp.float32)
        for r in range(16):
            for q in range(D // L):
                zero_v[r, pl.ds(q * L, L)] = zvec

        def zbody(i, carry):
            pltpu.async_copy(
                zero_v, acc_sh.at[pl.ds(s * ROWS_PER_TILE + i * 16, 16)],
                gsem0)
            return carry
        lax.fori_loop(0, ROWS_PER_TILE // 16, zbody, 0)
        pltpu.async_copy(
            zero_v.at[pl.ds(0, ROWS_PER_TILE % 16)],
            acc_sh.at[pl.ds(s * ROWS_PER_TILE + 16 * (ROWS_PER_TILE // 16),
                            ROWS_PER_TILE % 16)], gsem1)

        def zdrain(i, carry):
            pltpu.make_async_copy(
                zero_v, acc_sh.at[pl.ds(s * ROWS_PER_TILE, 16)], gsem0).wait()
            return carry
        lax.fori_loop(0, ROWS_PER_TILE // 16, zdrain, 0)
        pltpu.make_async_copy(
            zero_v.at[pl.ds(0, ROWS_PER_TILE % 16)],
            acc_sh.at[pl.ds(s * ROWS_PER_TILE, ROWS_PER_TILE % 16)],
            gsem1).wait()

    def fire_gather(g, p, sem):
        pltpu.async_copy(h1_hbm.at[src_v.at[g]], rows_v.at[p], sem)

    def drain_gather(g, p, sem):
        pltpu.make_async_copy(h1_hbm.at[src_v.at[g]],
                              rows_v.at[p], sem).wait()

    def fire_scatter(g, p, sem):
        pltpu.async_copy(rows_v.at[p], acc_sh.at[dst_v.at[g]], sem, add=True)

    def drain_scatter(g, p, sem):
        pltpu.make_async_copy(rows_v.at[p],
                              acc_sh.at[dst_v.at[g]], sem).wait()

    def run_round(base):
        pltpu.async_copy(edges_hbm.at[0, pl.ds(base, IB)], src_v, gsem0)
        pltpu.async_copy(edges_hbm.at[1, pl.ds(base, IB)], dst_v, gsem1)
        pltpu.make_async_copy(edges_hbm.at[0, pl.ds(base, IB)], src_v,
                              gsem0).wait()
        pltpu.make_async_copy(edges_hbm.at[1, pl.ds(base, IB)], dst_v,
                              gsem1).wait()

        fire_gather(0, 0, gsem0)
        fire_gather(1, 1, gsem1)

        def body(i, carry):
            g0 = 2 * i
            drain_gather(g0, 0, gsem0)
            fire_scatter(g0, 0, ssem0)
            drain_gather(g0 + 1, 1, gsem1)
            drain_scatter(g0, 0, ssem0)

            @pl.when(i < IB // 2 - 1)
            def _():
                fire_gather(g0 + 2, 0, gsem0)

            fire_scatter(g0 + 1, 1, ssem1)
            drain_scatter(g0 + 1, 1, ssem1)

            @pl.when(i < IB // 2 - 1)
            def _():
                fire_gather(g0 + 3, 1, gsem1)

            return carry
        lax.fori_loop(0, IB // 2, body, 0)

    zero_acc()
    plsc.subcore_barrier()
    wid = c * NS + s
    for r in range(CPW // IB):
        run_round(wid * CPW + r * IB)
    plsc.subcore_barrier()
    pltpu.sync_copy(acc_sh.at[pl.ds(s * ROWS_PER_TILE, ROWS_PER_TILE)],
                    out_hbm.at[c, pl.ds(s * ROWS_PER_TILE, ROWS_PER_TILE)])


def _fc_body(p_ref, norm_ref, w_ref, b_ref, h3_ref, sum_ref, sumsq_ref):
    i = pl.program_id(0)
    h = (p_ref[0] + p_ref[1]) * norm_ref[...]
    h = lax.dot_general(h, w_ref[...], (((1,), (1,)), ((), ())),
                        precision=lax.Precision.HIGHEST,
                        preferred_element_type=jnp.float32)
    h = jnp.maximum(h + b_ref[...][None, :], 0.0)
    h3_ref[...] = h

    @pl.when(i == 0)
    def _():
        sum_ref[...] = jnp.zeros_like(sum_ref)
        sumsq_ref[...] = jnp.zeros_like(sumsq_ref)

    sum_ref[...] += jnp.sum(h, axis=0, keepdims=True)
    sumsq_ref[...] += jnp.sum(h * h, axis=0, keepdims=True)


def _bn_body(h3_ref, sum_ref, sumsq_ref, gamma_ref, beta_ref, gid_ref,
             hbn_ref, phis_ref):
    i = pl.program_id(0)
    inv_n = 1.0 / N_NODES
    mean = sum_ref[...] * inv_n
    var = sumsq_ref[...] * inv_n - mean * mean
    scale = gamma_ref[...][None, :] / jnp.sqrt(var + 1e-5)
    hbn = (h3_ref[...] - mean) * scale + beta_ref[...][None, :]
    hbn_ref[...] = hbn

    gid = gid_ref[0]
    gids = lax.broadcasted_iota(jnp.int32, (NUM_GRAPHS, BLK), 0)
    onehot = (gids == gid).astype(jnp.float32)
    contrib = lax.dot_general(onehot, hbn, (((1,), (0,)), ((), ())),
                              precision=lax.Precision.HIGHEST,
                              preferred_element_type=jnp.float32)

    @pl.when(i == 0)
    def _():
        phis_ref[...] = jnp.zeros_like(phis_ref)

    phis_ref[...] += contrib


def kernel(x, edge_index, norm, graph_ids, W, b, gamma, beta):
    h1 = pl.pallas_call(
        _scale_body,
        grid=(NBLK,),
        in_specs=[pl.BlockSpec((BLK, D), lambda i: (i, 0)),
                  pl.BlockSpec((BLK, 1), lambda i: (i, 0))],
        out_specs=pl.BlockSpec((BLK, D), lambda i: (i, 0)),
        out_shape=jax.ShapeDtypeStruct((N_NODES, D), jnp.float32),
    )(x, norm)

    edges3 = edge_index.reshape(2, N_CHUNKS, CHUNK)

    mesh = plsc.VectorSubcoreMesh(core_axis_name="c", subcore_axis_name="s",
                                  num_cores=NC, num_subcores=NS)
    partials = pl.kernel(
        _edge_agg_body,
        out_type=jax.ShapeDtypeStruct((NC, ACC_ROWS, D), jnp.float32),
        mesh=mesh,
        scratch_types=[
            pltpu.VMEM((IB, CHUNK), jnp.int32),
            pltpu.VMEM((IB, CHUNK), jnp.int32),
            pltpu.VMEM((2, CHUNK, D), jnp.float32),
            pltpu.VMEM((16, D), jnp.float32),
            pltpu.VMEM_SHARED((ACC_ROWS, D), jnp.float32),
            pltpu.SemaphoreType.DMA,
            pltpu.SemaphoreType.DMA,
            pltpu.SemaphoreType.DMA,
            pltpu.SemaphoreType.DMA,
        ],
    )(h1, edges3)

    h3, colsum, colsumsq = pl.pallas_call(
        _fc_body,
        grid=(NBLK,),
        in_specs=[pl.BlockSpec((NC, BLK, D), lambda i: (0, i, 0)),
                  pl.BlockSpec((BLK, 1), lambda i: (i, 0)),
                  pl.BlockSpec((D, D), lambda i: (0, 0)),
                  pl.BlockSpec((D,), lambda i: (0,))],
        out_specs=[pl.BlockSpec((BLK, D), lambda i: (i, 0)),
                   pl.BlockSpec((1, D), lambda i: (0, 0)),
                   pl.BlockSpec((1, D), lambda i: (0, 0))],
        out_shape=[jax.ShapeDtypeStruct((N_NODES, D), jnp.float32),
                   jax.ShapeDtypeStruct((1, D), jnp.float32),
                   jax.ShapeDtypeStruct((1, D), jnp.float32)],
    )(partials, norm, W, b)

    hbn, phis = pl.pallas_call(
        _bn_body,
        grid=(NBLK,),
        in_specs=[pl.BlockSpec((BLK, D), lambda i: (i, 0)),
                  pl.BlockSpec((1, D), lambda i: (0, 0)),
                  pl.BlockSpec((1, D), lambda i: (0, 0)),
                  pl.BlockSpec((D,), lambda i: (0,)),
                  pl.BlockSpec((D,), lambda i: (0,)),
                  pl.BlockSpec((1, 1, BLK), lambda i: (i, 0, 0))],
        out_specs=[pl.BlockSpec((BLK, D), lambda i: (i, 0)),
                   pl.BlockSpec((NUM_GRAPHS, D), lambda i: (0, 0))],
        out_shape=[jax.ShapeDtypeStruct((N_NODES, D), jnp.float32),
                   jax.ShapeDtypeStruct((NUM_GRAPHS, D), jnp.float32)],
    )(h3, colsum, colsumsq, gamma, beta,
      graph_ids.reshape(NBLK, 1, BLK))

    return (hbn, phis)


# speedup vs baseline: 4.1517x; 1.0279x over previous
import functools

import jax
import jax.numpy as jnp
from jax import lax
from jax.experimental import pallas as pl
from jax.experimental.pallas import tpu as pltpu
from jax.experimental.pallas import tpu_sc as plsc

N_NODES = 10000
N_EDGES = 320000
D = 128
NUM_GRAPHS = 64

NC, NS, L = 2, 16, 16
NW = NC * NS
CHUNK = 125
N_CHUNKS = N_EDGES // CHUNK
CPW = N_CHUNKS // NW
IB = 40
ROWS_PER_TILE = 632
ACC_ROWS = ROWS_PER_TILE * NS

BLK = 1000
NBLK = N_NODES // BLK


def _scale_body(x_ref, norm_ref, o_ref):
    o_ref[...] = x_ref[...] * norm_ref[...]


def _edge_agg_body(h1_hbm, edges_hbm, out_hbm,
                   src_v, dst_v, rows_v, zero_v, acc_sh,
                   gsem0, gsem1, ssem0, ssem1):
    c = lax.axis_index("c")
    s = lax.axis_index("s")

    def zero_acc():
        zvec = jnp.zeros((L,), jnp.float32)
        for r in range(16):
            for q in range(D // L):
                zero_v[r, pl.ds(q * L, L)] = zvec

        def zbody(i, carry):
            pltpu.async_copy(
                zero_v, acc_sh.at[pl.ds(s * ROWS_PER_TILE + i * 16, 16)],
                gsem0)
            return carry
        lax.fori_loop(0, ROWS_PER_TILE // 16, zbody, 0)
        pltpu.async_copy(
            zero_v.at[pl.ds(0, ROWS_PER_TILE % 16)],
            acc_sh.at[pl.ds(s * ROWS_PER_TILE + 16 * (ROWS_PER_TILE // 16),
                            ROWS_PER_TILE % 16)], gsem1)

        def zdrain(i, carry):
            pltpu.make_async_copy(
                zero_v, acc_sh.at[pl.ds(s * ROWS_PER_TILE, 16)], gsem0).wait()
            return carry
        lax.fori_loop(0, ROWS_PER_TILE // 16, zdrain, 0)
        pltpu.make_async_copy(
            zero_v.at[pl.ds(0, ROWS_PER_TILE % 16)],
            acc_sh.at[pl.ds(s * ROWS_PER_TILE, ROWS_PER_TILE % 16)],
            gsem1).wait()

    def fire_gather(g, p, sem):
        pltpu.async_copy(h1_hbm.at[src_v.at[g]], rows_v.at[p], sem)

    def drain_gather(g, p, sem):
        pltpu.make_async_copy(h1_hbm.at[src_v.at[g]],
                              rows_v.at[p], sem).wait()

    def fire_scatter(g, p, sem):
        pltpu.async_copy(rows_v.at[p], acc_sh.at[dst_v.at[g]], sem, add=True)

    def drain_scatter(g, p, sem):
        pltpu.make_async_copy(rows_v.at[p],
                              acc_sh.at[dst_v.at[g]], sem).wait()

    def run_round(base):
        pltpu.async_copy(edges_hbm.at[0, pl.ds(base, IB)], src_v, gsem0)
        pltpu.async_copy(edges_hbm.at[1, pl.ds(base, IB)], dst_v, gsem1)
        pltpu.make_async_copy(edges_hbm.at[0, pl.ds(base, IB)], src_v,
                              gsem0).wait()
        pltpu.make_async_copy(edges_hbm.at[1, pl.ds(base, IB)], dst_v,
                              gsem1).wait()

        fire_gather(0, 0, gsem0)
        fire_gather(1, 1, gsem1)

        def body(i, carry):
            g0 = 2 * i
            drain_gather(g0, 0, gsem0)
            fire_scatter(g0, 0, ssem0)
            drain_gather(g0 + 1, 1, gsem1)
            drain_scatter(g0, 0, ssem0)

            @pl.when(i < IB // 2 - 1)
            def _():
                fire_gather(g0 + 2, 0, gsem0)

            fire_scatter(g0 + 1, 1, ssem1)
            drain_scatter(g0 + 1, 1, ssem1)

            @pl.when(i < IB // 2 - 1)
            def _():
                fire_gather(g0 + 3, 1, gsem1)

            return carry
        lax.fori_loop(0, IB // 2, body, 0)

    zero_acc()
    plsc.subcore_barrier()
    wid = c * NS + s
    for r in range(CPW // IB):
        run_round(wid * CPW + r * IB)
    plsc.subcore_barrier()
    pltpu.sync_copy(acc_sh.at[pl.ds(s * ROWS_PER_TILE, ROWS_PER_TILE)],
                    out_hbm.at[c, pl.ds(s * ROWS_PER_TILE, ROWS_PER_TILE)])


def _post_body(p_ref, norm_ref, w_ref, b_ref, gamma_ref, beta_ref, gid_ref,
               hbn_ref, phis_ref, h3_scr, sum_scr, sumsq_scr):
    ph = pl.program_id(0)
    i = pl.program_id(1)

    @pl.when(ph == 0)
    def _():
        h = (p_ref[0] + p_ref[1]) * norm_ref[...]
        h = lax.dot_general(h, w_ref[...], (((1,), (1,)), ((), ())),
                            precision=lax.Precision.HIGHEST,
                            preferred_element_type=jnp.float32)
        h = jnp.maximum(h + b_ref[...][None, :], 0.0)
        h3_scr[pl.ds(i * BLK, BLK), :] = h

        @pl.when(i == 0)
        def _():
            sum_scr[...] = jnp.zeros_like(sum_scr)
            sumsq_scr[...] = jnp.zeros_like(sumsq_scr)

        sum_scr[...] += jnp.sum(h, axis=0, keepdims=True)
        sumsq_scr[...] += jnp.sum(h * h, axis=0, keepdims=True)

    @pl.when(ph == 1)
    def _():
        inv_n = 1.0 / N_NODES
        mean = sum_scr[...] * inv_n
        var = sumsq_scr[...] * inv_n - mean * mean
        scale = gamma_ref[...][None, :] / jnp.sqrt(var + 1e-5)
        hbn = (h3_scr[pl.ds(i * BLK, BLK), :] - mean) * scale \
            + beta_ref[...][None, :]
        hbn_ref[...] = hbn

        gid = gid_ref[0]
        gids = lax.broadcasted_iota(jnp.int32, (NUM_GRAPHS, BLK), 0)
        onehot = (gids == gid).astype(jnp.float32)
        contrib = lax.dot_general(onehot, hbn, (((1,), (0,)), ((), ())),
                                  precision=lax.Precision.HIGHEST,
                                  preferred_element_type=jnp.float32)

        @pl.when(i == 0)
        def _():
            phis_ref[...] = jnp.zeros_like(phis_ref)

        phis_ref[...] += contrib


def kernel(x, edge_index, norm, graph_ids, W, b, gamma, beta):
    h1 = pl.pallas_call(
        _scale_body,
        grid=(NBLK,),
        in_specs=[pl.BlockSpec((BLK, D), lambda i: (i, 0)),
                  pl.BlockSpec((BLK, 1), lambda i: (i, 0))],
        out_specs=pl.BlockSpec((BLK, D), lambda i: (i, 0)),
        out_shape=jax.ShapeDtypeStruct((N_NODES, D), jnp.float32),
    )(x, norm)

    edges3 = edge_index.reshape(2, N_CHUNKS, CHUNK)

    mesh = plsc.VectorSubcoreMesh(core_axis_name="c", subcore_axis_name="s",
                                  num_cores=NC, num_subcores=NS)
    partials = pl.kernel(
        _edge_agg_body,
        out_type=jax.ShapeDtypeStruct((NC, ACC_ROWS, D), jnp.float32),
        mesh=mesh,
        scratch_types=[
            pltpu.VMEM((IB, CHUNK), jnp.int32),
            pltpu.VMEM((IB, CHUNK), jnp.int32),
            pltpu.VMEM((2, CHUNK, D), jnp.float32),
            pltpu.VMEM((16, D), jnp.float32),
            pltpu.VMEM_SHARED((ACC_ROWS, D), jnp.float32),
            pltpu.SemaphoreType.DMA,
            pltpu.SemaphoreType.DMA,
            pltpu.SemaphoreType.DMA,
            pltpu.SemaphoreType.DMA,
        ],
    )(h1, edges3)

    hbn, phis = pl.pallas_call(
        _post_body,
        grid=(2, NBLK),
        in_specs=[
            pl.BlockSpec((NC, BLK, D),
                         lambda ph, i: (0, jnp.where(ph == 0, i, 0), 0)),
            pl.BlockSpec((BLK, 1),
                         lambda ph, i: (jnp.where(ph == 0, i, 0), 0)),
            pl.BlockSpec((D, D), lambda ph, i: (0, 0)),
            pl.BlockSpec((D,), lambda ph, i: (0,)),
            pl.BlockSpec((D,), lambda ph, i: (0,)),
            pl.BlockSpec((D,), lambda ph, i: (0,)),
            pl.BlockSpec((1, 1, BLK),
                         lambda ph, i: (jnp.where(ph == 1, i, 0), 0, 0)),
        ],
        out_specs=[
            pl.BlockSpec((BLK, D),
                         lambda ph, i: (jnp.where(ph == 1, i, 0), 0)),
            pl.BlockSpec((NUM_GRAPHS, D), lambda ph, i: (0, 0)),
        ],
        out_shape=[jax.ShapeDtypeStruct((N_NODES, D), jnp.float32),
                   jax.ShapeDtypeStruct((NUM_GRAPHS, D), jnp.float32)],
        scratch_shapes=[pltpu.VMEM((N_NODES, D), jnp.float32),
                        pltpu.VMEM((1, D), jnp.float32),
                        pltpu.VMEM((1, D), jnp.float32)],
    )(partials, norm, W, b, gamma, beta, graph_ids.reshape(NBLK, 1, BLK))

    return (hbn, phis)


# speedup vs baseline: 4.3786x; 1.0546x over previous
import functools

import jax
import jax.numpy as jnp
from jax import lax
from jax.experimental import pallas as pl
from jax.experimental.pallas import tpu as pltpu
from jax.experimental.pallas import tpu_sc as plsc

N_NODES = 10000
N_EDGES = 320000
D = 128
NUM_GRAPHS = 64

NC, NS, L = 2, 16, 16
NW = NC * NS
CHUNK = 125
N_CHUNKS = N_EDGES // CHUNK
CPW = N_CHUNKS // NW
IB = 40
ROWS_PER_TILE = 632
ACC_ROWS = ROWS_PER_TILE * NS

BLK = 2000
NBLK = N_NODES // BLK


def _scale_body(x_ref, norm_ref, o_ref):
    o_ref[...] = x_ref[...] * norm_ref[...]


def _edge_agg_body(h1_hbm, edges_hbm, out_hbm,
                   src_v, dst_v, rows_v, zero_v, acc_sh,
                   gsem0, gsem1, ssem0, ssem1):
    c = lax.axis_index("c")
    s = lax.axis_index("s")

    def zero_acc():
        zvec = jnp.zeros((L,), jnp.float32)
        for r in range(16):
            for q in range(D // L):
                zero_v[r, pl.ds(q * L, L)] = zvec

        def zbody(i, carry):
            pltpu.async_copy(
                zero_v, acc_sh.at[pl.ds(s * ROWS_PER_TILE + i * 16, 16)],
                gsem0)
            return carry
        lax.fori_loop(0, ROWS_PER_TILE // 16, zbody, 0)
        pltpu.async_copy(
            zero_v.at[pl.ds(0, ROWS_PER_TILE % 16)],
            acc_sh.at[pl.ds(s * ROWS_PER_TILE + 16 * (ROWS_PER_TILE // 16),
                            ROWS_PER_TILE % 16)], gsem1)

        def zdrain(i, carry):
            pltpu.make_async_copy(
                zero_v, acc_sh.at[pl.ds(s * ROWS_PER_TILE, 16)], gsem0).wait()
            return carry
        lax.fori_loop(0, ROWS_PER_TILE // 16, zdrain, 0)
        pltpu.make_async_copy(
            zero_v.at[pl.ds(0, ROWS_PER_TILE % 16)],
            acc_sh.at[pl.ds(s * ROWS_PER_TILE, ROWS_PER_TILE % 16)],
            gsem1).wait()

    def fire_gather(g, p, sem):
        pltpu.async_copy(h1_hbm.at[src_v.at[g]], rows_v.at[p], sem)

    def drain_gather(g, p, sem):
        pltpu.make_async_copy(h1_hbm.at[src_v.at[g]],
                              rows_v.at[p], sem).wait()

    def fire_scatter(g, p, sem):
        pltpu.async_copy(rows_v.at[p], acc_sh.at[dst_v.at[g]], sem, add=True)

    def drain_scatter(g, p, sem):
        pltpu.make_async_copy(rows_v.at[p],
                              acc_sh.at[dst_v.at[g]], sem).wait()

    def load_idx(base):
        pltpu.async_copy(edges_hbm.at[0, pl.ds(base, IB)], src_v, ssem0)
        pltpu.async_copy(edges_hbm.at[1, pl.ds(base, IB)], dst_v, ssem1)
        pltpu.make_async_copy(edges_hbm.at[0, pl.ds(base, IB)], src_v,
                              ssem0).wait()
        pltpu.make_async_copy(edges_hbm.at[1, pl.ds(base, IB)], dst_v,
                              ssem1).wait()

    def run_round(base, first):
        if not first:
            load_idx(base)
            fire_gather(0, 0, gsem0)
            fire_gather(1, 1, gsem1)

        def body(i, carry):
            g0 = 2 * i
            drain_gather(g0, 0, gsem0)
            fire_scatter(g0, 0, ssem0)
            drain_gather(g0 + 1, 1, gsem1)
            drain_scatter(g0, 0, ssem0)

            @pl.when(i < IB // 2 - 1)
            def _():
                fire_gather(g0 + 2, 0, gsem0)

            fire_scatter(g0 + 1, 1, ssem1)
            drain_scatter(g0 + 1, 1, ssem1)

            @pl.when(i < IB // 2 - 1)
            def _():
                fire_gather(g0 + 3, 1, gsem1)

            return carry
        lax.fori_loop(0, IB // 2, body, 0)

    wid = c * NS + s
    zero_acc()
    load_idx(wid * CPW)
    fire_gather(0, 0, gsem0)
    fire_gather(1, 1, gsem1)
    plsc.subcore_barrier()
    for r in range(CPW // IB):
        run_round(wid * CPW + r * IB, r == 0)
    plsc.subcore_barrier()
    pltpu.sync_copy(acc_sh.at[pl.ds(s * ROWS_PER_TILE, ROWS_PER_TILE)],
                    out_hbm.at[c, pl.ds(s * ROWS_PER_TILE, ROWS_PER_TILE)])


def _post_body(p_ref, norm_ref, w_ref, b_ref, gamma_ref, beta_ref, gid_ref,
               hbn_ref, phis_ref, h3_scr, sum_scr, sumsq_scr):
    ph = pl.program_id(0)
    i = pl.program_id(1)

    @pl.when(ph == 0)
    def _():
        h = (p_ref[0] + p_ref[1]) * norm_ref[...]
        h = lax.dot_general(h, w_ref[...], (((1,), (1,)), ((), ())),
                            precision=lax.Precision.HIGHEST,
                            preferred_element_type=jnp.float32)
        h = jnp.maximum(h + b_ref[...][None, :], 0.0)
        h3_scr[pl.ds(i * BLK, BLK), :] = h

        @pl.when(i == 0)
        def _():
            sum_scr[...] = jnp.zeros_like(sum_scr)
            sumsq_scr[...] = jnp.zeros_like(sumsq_scr)

        sum_scr[...] += jnp.sum(h, axis=0, keepdims=True)
        sumsq_scr[...] += jnp.sum(h * h, axis=0, keepdims=True)

    @pl.when(ph == 1)
    def _():
        inv_n = 1.0 / N_NODES
        mean = sum_scr[...] * inv_n
        var = sumsq_scr[...] * inv_n - mean * mean
        scale = gamma_ref[...][None, :] / jnp.sqrt(var + 1e-5)
        hbn = (h3_scr[pl.ds(i * BLK, BLK), :] - mean) * scale \
            + beta_ref[...][None, :]
        hbn_ref[...] = hbn

        gid = gid_ref[0]
        gids = lax.broadcasted_iota(jnp.int32, (NUM_GRAPHS, BLK), 0)
        onehot = (gids == gid).astype(jnp.float32)
        contrib = lax.dot_general(onehot, hbn, (((1,), (0,)), ((), ())),
                                  precision=lax.Precision.HIGHEST,
                                  preferred_element_type=jnp.float32)

        @pl.when(i == 0)
        def _():
            phis_ref[...] = jnp.zeros_like(phis_ref)

        phis_ref[...] += contrib


def kernel(x, edge_index, norm, graph_ids, W, b, gamma, beta):
    h1 = pl.pallas_call(
        _scale_body,
        grid=(NBLK,),
        in_specs=[pl.BlockSpec((BLK, D), lambda i: (i, 0)),
                  pl.BlockSpec((BLK, 1), lambda i: (i, 0))],
        out_specs=pl.BlockSpec((BLK, D), lambda i: (i, 0)),
        out_shape=jax.ShapeDtypeStruct((N_NODES, D), jnp.float32),
    )(x, norm)

    edges3 = edge_index.reshape(2, N_CHUNKS, CHUNK)

    mesh = plsc.VectorSubcoreMesh(core_axis_name="c", subcore_axis_name="s",
                                  num_cores=NC, num_subcores=NS)
    partials = pl.kernel(
        _edge_agg_body,
        out_type=jax.ShapeDtypeStruct((NC, ACC_ROWS, D), jnp.float32),
        mesh=mesh,
        scratch_types=[
            pltpu.VMEM((IB, CHUNK), jnp.int32),
            pltpu.VMEM((IB, CHUNK), jnp.int32),
            pltpu.VMEM((2, CHUNK, D), jnp.float32),
            pltpu.VMEM((16, D), jnp.float32),
            pltpu.VMEM_SHARED((ACC_ROWS, D), jnp.float32),
            pltpu.SemaphoreType.DMA,
            pltpu.SemaphoreType.DMA,
            pltpu.SemaphoreType.DMA,
            pltpu.SemaphoreType.DMA,
        ],
    )(h1, edges3)

    hbn, phis = pl.pallas_call(
        _post_body,
        grid=(2, NBLK),
        in_specs=[
            pl.BlockSpec((NC, BLK, D),
                         lambda ph, i: (0, jnp.where(ph == 0, i, 0), 0)),
            pl.BlockSpec((BLK, 1),
                         lambda ph, i: (jnp.where(ph == 0, i, 0), 0)),
            pl.BlockSpec((D, D), lambda ph, i: (0, 0)),
            pl.BlockSpec((D,), lambda ph, i: (0,)),
            pl.BlockSpec((D,), lambda ph, i: (0,)),
            pl.BlockSpec((D,), lambda ph, i: (0,)),
            pl.BlockSpec((1, 1, BLK),
                         lambda ph, i: (jnp.where(ph == 1, i, 0), 0, 0)),
        ],
        out_specs=[
            pl.BlockSpec((BLK, D),
                         lambda ph, i: (jnp.where(ph == 1, i, 0), 0)),
            pl.BlockSpec((NUM_GRAPHS, D), lambda ph, i: (0, 0)),
        ],
        out_shape=[jax.ShapeDtypeStruct((N_NODES, D), jnp.float32),
                   jax.ShapeDtypeStruct((NUM_GRAPHS, D), jnp.float32)],
        scratch_shapes=[pltpu.VMEM((N_NODES, D), jnp.float32),
                        pltpu.VMEM((1, D), jnp.float32),
                        pltpu.VMEM((1, D), jnp.float32)],
    )(partials, norm, W, b, gamma, beta, graph_ids.reshape(NBLK, 1, BLK))

    return (hbn, phis)
